# Initial kernel scaffold; baseline (speedup 1.0000x reference)
import functools

import jax
import jax.numpy as jnp
from jax import lax
from jax.experimental import pallas as pl
from jax.experimental.pallas import tpu as pltpu
from jax.experimental.pallas import tpu_sc as plsc

N = 10000
E = 320000
D = 128
H = 128
B = 128
MLP_DIM = 300
DEPTH = 3

NC = 2
NS = 16
CHUNK = 128
NCHUNK = 158
EPT = NCHUNK * CHUNK
TRASH = N
NP = 10240
BLK = 1024


@functools.cache
def _sc_seg_call():
    mesh = plsc.VectorSubcoreMesh(core_axis_name="c", subcore_axis_name="s")

    @functools.partial(
        pl.kernel,
        out_type=jax.ShapeDtypeStruct((NC, NP, H), jnp.float32),
        mesh=mesh,
        scratch_types=[
            pltpu.VMEM((NCHUNK, CHUNK), jnp.int32),
            pltpu.VMEM((NCHUNK, CHUNK), jnp.int32),
            pltpu.VMEM((2, CHUNK, H), jnp.float32),
            pltpu.VMEM_SHARED((NP, H), jnp.float32),
            pltpu.SemaphoreType.DMA,
            pltpu.SemaphoreType.DMA,
        ],
    )
    def sc_seg(h_hbm, srci_hbm, dsti_hbm, zeros_hbm, out_hbm,
               srcv, dstv, rows, acc, sem0, sem1):
        cid = lax.axis_index("c")
        sid = lax.axis_index("s")
        h_c = h_hbm.at[cid]
        pltpu.sync_copy(srci_hbm.at[cid, sid], srcv)
        pltpu.sync_copy(dsti_hbm.at[cid, sid], dstv)
        rpt = NP // NS
        pltpu.sync_copy(zeros_hbm.at[pl.ds(sid * rpt, rpt)],
                        acc.at[pl.ds(sid * rpt, rpt)])
        plsc.subcore_barrier()

        sems = (sem0, sem1)

        def gather(j, buf):
            return pltpu.make_async_copy(h_c.at[srcv.at[j]], rows.at[buf],
                                         sems[buf])

        gather(0, 0).start()

        @pl.loop(0, NCHUNK, step=2)
        def _(j):
            gather(j, 0).wait()
            gather(j + 1, 1).start()
            pltpu.sync_copy(rows.at[0], acc.at[dstv.at[j]], add=True)
            gather(j + 1, 1).wait()

            @pl.when(j + 2 < NCHUNK)
            def _():
                gather(j + 2, 0).start()

            pltpu.sync_copy(rows.at[1], acc.at[dstv.at[j + 1]], add=True)

        plsc.subcore_barrier()
        pltpu.sync_copy(acc.at[pl.ds(sid * rpt, rpt)],
                        out_hbm.at[cid].at[pl.ds(sid * rpt, rpt)])

    return sc_seg


def _segsum_both(h, srci, dsti, zeros):
    return _sc_seg_call()(h, srci, dsti, zeros)


def _dot(a, b):
    return lax.dot_general(a, b, (((1,), (0,)), ((), ())),
                           preferred_element_type=jnp.float32)


def _tc_in_body(x_ref, w_ref, o_ref):
    o_ref[0] = jnp.maximum(_dot(x_ref[0], w_ref[0]), 0.0)


def _tc_in(x, wi):
    return pl.pallas_call(
        _tc_in_body,
        grid=(NC, NP // BLK),
        in_specs=[pl.BlockSpec((1, BLK, D), lambda m, i: (m, i, 0)),
                  pl.BlockSpec((1, D, H), lambda m, i: (m, 0, 0))],
        out_specs=pl.BlockSpec((1, BLK, H), lambda m, i: (m, i, 0)),
        out_shape=jax.ShapeDtypeStruct((NC, NP, H), jnp.float32),
    )(x, wi)


def _tc_update_body(m_ref, h0_ref, w_ref, o_ref):
    o_ref[0] = jnp.maximum(h0_ref[0] + _dot(m_ref[0], w_ref[0]), 0.0)


def _tc_update(m, h0, wm):
    return pl.pallas_call(
        _tc_update_body,
        grid=(NC, NP // BLK),
        in_specs=[pl.BlockSpec((1, BLK, H), lambda m, i: (m, i, 0)),
                  pl.BlockSpec((1, BLK, H), lambda m, i: (m, i, 0)),
                  pl.BlockSpec((1, H, H), lambda m, i: (m, 0, 0))],
        out_specs=pl.BlockSpec((1, BLK, H), lambda m, i: (m, i, 0)),
        out_shape=jax.ShapeDtypeStruct((NC, NP, H), jnp.float32),
    )(m, h0, wm)


def _tc_readout_body(h_ref, ids_ref, o_ref, sacc, cacc):
    i = pl.program_id(1)

    @pl.when(i == 0)
    def _():
        sacc[...] = jnp.zeros((B, H), jnp.float32)
        cacc[...] = jnp.zeros((B, H), jnp.float32)

    oh = (ids_ref[...] == lax.broadcasted_iota(jnp.int32, (BLK, B), 1))
    oh = oh.astype(jnp.float32)
    contract = (((0,), (0,)), ((), ()))
    sacc[...] += lax.dot_general(oh, h_ref[0], contract,
                                 preferred_element_type=jnp.float32)
    cacc[...] += lax.dot_general(oh, jnp.ones((BLK, H), jnp.float32), contract,
                                 preferred_element_type=jnp.float32)

    @pl.when(i == NP // BLK - 1)
    def _():
        o_ref[0] = sacc[...] / jnp.maximum(cacc[...], 1.0)


def _tc_readout(h, ids):
    return pl.pallas_call(
        _tc_readout_body,
        grid=(NC, NP // BLK),
        in_specs=[pl.BlockSpec((1, BLK, H), lambda m, i: (m, i, 0)),
                  pl.BlockSpec((BLK, 1), lambda m, i: (i, 0))],
        out_specs=pl.BlockSpec((1, B, H), lambda m, i: (m, 0, 0)),
        out_shape=jax.ShapeDtypeStruct((NC, B, H), jnp.float32),
        scratch_shapes=[pltpu.VMEM((B, H), jnp.float32),
                        pltpu.VMEM((B, H), jnp.float32)],
    )(h, ids)


def _tc_mlp_body(v_ref, w1a_ref, w1b_ref, b1_ref, w2_ref, b2_ref, w3_ref,
                 b3_ref, o_ref):
    h1 = jnp.maximum(_dot(v_ref[0], w1a_ref[...]) +
                     _dot(v_ref[1], w1b_ref[...]) + b1_ref[...], 0.0)
    h2 = jnp.maximum(_dot(h1, w2_ref[...]) + b2_ref[...], 0.0)
    o_ref[...] = jnp.sum(h2 * w3_ref[...], axis=1, keepdims=True) + b3_ref[...]


def _tc_mlp(vec, w1, b1, w2, b2, w3, b3):
    return pl.pallas_call(
        _tc_mlp_body,
        out_shape=jax.ShapeDtypeStruct((B, 1), jnp.float32),
    )(vec, w1[:H], w1[H:], b1.reshape(1, MLP_DIM), w2,
      b2.reshape(1, MLP_DIM), w3.reshape(1, MLP_DIM), b3.reshape(1, 1))


def _prep_edges(ei_sol, ei_solv):
    def prep(row, fill):
        r = row.reshape(NS, E // NS)
        r = jnp.pad(r, ((0, 0), (0, EPT - E // NS)), constant_values=fill)
        return r.reshape(NS, NCHUNK, CHUNK)

    srci = jnp.stack([prep(ei_sol[0], 0), prep(ei_solv[0], 0)])
    dsti = jnp.stack([prep(ei_sol[1], TRASH), prep(ei_solv[1], TRASH)])
    return srci, dsti


def kernel(x_solute, x_solvent, edge_index_solute, edge_index_solvent,
           graph_ids, W_in_solute, W_msg_solute, W_in_solvent, W_msg_solvent,
           W1, b1, W2, b2, W3, b3):
    x_both = jnp.stack([x_solute, x_solvent])
    x_both = jnp.pad(x_both, ((0, 0), (0, NP - N), (0, 0)))
    wi = jnp.stack([W_in_solute, W_in_solvent])
    wm = jnp.stack([W_msg_solute, W_msg_solvent])
    srci, dsti = _prep_edges(edge_index_solute, edge_index_solvent)
    zeros = jnp.zeros((NP, H), jnp.float32)
    ids = jnp.pad(graph_ids, (0, NP - N), constant_values=B).reshape(NP, 1)

    h0 = _tc_in(x_both, wi)
    h = h0
    for _ in range(DEPTH):
        m = _segsum_both(h, srci, dsti, zeros)
        h = _tc_update(m, h0, wm)
    vec = _tc_readout(h, ids)
    return _tc_mlp(vec, W1, b1, W2, b2, W3, b3)


# speedup vs baseline: 3.3039x; 3.3039x over previous
import functools

import jax
import jax.numpy as jnp
from jax import lax
from jax.experimental import pallas as pl
from jax.experimental.pallas import tpu as pltpu
from jax.experimental.pallas import tpu_sc as plsc

N = 10000
E = 320000
D = 128
H = 128
B = 128
MLP_DIM = 300
DEPTH = 3

NC = 2
NS = 16
CHUNK = 128
NCHUNK = 158
EPT = NCHUNK * CHUNK
TRASH = N
NP = 10240
BLK = 1024


HH = H // 2


@functools.cache
def _sc_seg_call():
    mesh = plsc.VectorSubcoreMesh(core_axis_name="c", subcore_axis_name="s")

    @functools.partial(
        pl.kernel,
        out_type=jax.ShapeDtypeStruct((NC, NP, 2, HH), jnp.float32),
        mesh=mesh,
        compiler_params=pltpu.CompilerParams(use_tc_tiling_on_sc=False),
        scratch_types=[
            pltpu.VMEM((2, NCHUNK, CHUNK), jnp.int32),
            pltpu.VMEM((NCHUNK, CHUNK), jnp.int32),
            pltpu.VMEM((2, CHUNK, HH), jnp.float32),
            pltpu.VMEM_SHARED((NP, HH), jnp.float32),
            pltpu.SemaphoreType.DMA,
            pltpu.SemaphoreType.DMA,
        ],
    )
    def sc_seg(h_hbm, srci_hbm, dsti_hbm, zeros_hbm, out_hbm,
               srcv, dstv, rows, acc, sem0, sem1):
        cid = lax.axis_index("c")
        sid = lax.axis_index("s")
        h_v = h_hbm.at[cid]
        out_v = out_hbm.at[cid]
        zeros_v = zeros_hbm
        pltpu.sync_copy(srci_hbm.at[cid, sid], srcv)
        pltpu.sync_copy(dsti_hbm.at[cid, sid], dstv)
        rpt = NP // NS
        sems = (sem0, sem1)

        for half in range(2):
            pltpu.sync_copy(zeros_v.at[pl.ds(sid * rpt, rpt)],
                            acc.at[pl.ds(sid * rpt, rpt)])
            plsc.subcore_barrier()

            def gather(j, buf):
                return pltpu.make_async_copy(h_v.at[srcv.at[half, j]],
                                             rows.at[buf], sems[buf])

            gather(0, 0).start()

            @pl.loop(0, NCHUNK, step=2)
            def _(j):
                gather(j, 0).wait()
                gather(j + 1, 1).start()
                pltpu.sync_copy(rows.at[0], acc.at[dstv.at[j]], add=True)
                gather(j + 1, 1).wait()

                @pl.when(j + 2 < NCHUNK)
                def _():
                    gather(j + 2, 0).start()

                pltpu.sync_copy(rows.at[1], acc.at[dstv.at[j + 1]], add=True)

            plsc.subcore_barrier()
            pltpu.sync_copy(acc.at[pl.ds(sid * rpt, rpt)],
                            out_v.at[pl.ds(sid * rpt, rpt), half])

    return sc_seg


def _segsum_both(h, srci2, dsti, zeros):
    h2 = h.reshape(NC, 2 * NP, HH)
    m = _sc_seg_call()(h2, srci2, dsti, zeros)
    return m.reshape(NC, NP, H)


def _dot(a, b):
    return lax.dot_general(a, b, (((1,), (0,)), ((), ())),
                           precision=lax.Precision.HIGHEST,
                           preferred_element_type=jnp.float32)


def _tc_in_body(x_ref, w_ref, o_ref):
    o_ref[0] = jnp.maximum(_dot(x_ref[0], w_ref[0]), 0.0)


def _tc_in(x, wi):
    return pl.pallas_call(
        _tc_in_body,
        grid=(NC, NP // BLK),
        in_specs=[pl.BlockSpec((1, BLK, D), lambda m, i: (m, i, 0)),
                  pl.BlockSpec((1, D, H), lambda m, i: (m, 0, 0))],
        out_specs=pl.BlockSpec((1, BLK, H), lambda m, i: (m, i, 0)),
        out_shape=jax.ShapeDtypeStruct((NC, NP, H), jnp.float32),
    )(x, wi)


def _tc_update_body(m_ref, h0_ref, w_ref, o_ref):
    o_ref[0] = jnp.maximum(h0_ref[0] + _dot(m_ref[0], w_ref[0]), 0.0)


def _tc_update(m, h0, wm):
    return pl.pallas_call(
        _tc_update_body,
        grid=(NC, NP // BLK),
        in_specs=[pl.BlockSpec((1, BLK, H), lambda m, i: (m, i, 0)),
                  pl.BlockSpec((1, BLK, H), lambda m, i: (m, i, 0)),
                  pl.BlockSpec((1, H, H), lambda m, i: (m, 0, 0))],
        out_specs=pl.BlockSpec((1, BLK, H), lambda m, i: (m, i, 0)),
        out_shape=jax.ShapeDtypeStruct((NC, NP, H), jnp.float32),
    )(m, h0, wm)


def _tc_readout_body(h_ref, ids_ref, o_ref, sacc, cacc):
    i = pl.program_id(1)

    @pl.when(i == 0)
    def _():
        sacc[...] = jnp.zeros((B, H), jnp.float32)
        cacc[...] = jnp.zeros((B, H), jnp.float32)

    oh = (ids_ref[...] == lax.broadcasted_iota(jnp.int32, (BLK, B), 1))
    oh = oh.astype(jnp.float32)
    contract = (((0,), (0,)), ((), ()))
    sacc[...] += lax.dot_general(oh, h_ref[0], contract,
                                 precision=lax.Precision.HIGHEST,
                                 preferred_element_type=jnp.float32)
    cacc[...] += lax.dot_general(oh, jnp.ones((BLK, H), jnp.float32), contract,
                                 precision=lax.Precision.HIGHEST,
                                 preferred_element_type=jnp.float32)

    @pl.when(i == NP // BLK - 1)
    def _():
        o_ref[0] = sacc[...] / jnp.maximum(cacc[...], 1.0)


def _tc_readout(h, ids):
    return pl.pallas_call(
        _tc_readout_body,
        grid=(NC, NP // BLK),
        in_specs=[pl.BlockSpec((1, BLK, H), lambda m, i: (m, i, 0)),
                  pl.BlockSpec((BLK, 1), lambda m, i: (i, 0))],
        out_specs=pl.BlockSpec((1, B, H), lambda m, i: (m, 0, 0)),
        out_shape=jax.ShapeDtypeStruct((NC, B, H), jnp.float32),
        scratch_shapes=[pltpu.VMEM((B, H), jnp.float32),
                        pltpu.VMEM((B, H), jnp.float32)],
    )(h, ids)


def _tc_mlp_body(v_ref, w1a_ref, w1b_ref, b1_ref, w2_ref, b2_ref, w3_ref,
                 b3_ref, o_ref):
    h1 = jnp.maximum(_dot(v_ref[0], w1a_ref[...]) +
                     _dot(v_ref[1], w1b_ref[...]) + b1_ref[...], 0.0)
    h2 = jnp.maximum(_dot(h1, w2_ref[...]) + b2_ref[...], 0.0)
    o_ref[...] = jnp.sum(h2 * w3_ref[...], axis=1, keepdims=True) + b3_ref[...]


def _tc_mlp(vec, w1, b1, w2, b2, w3, b3):
    return pl.pallas_call(
        _tc_mlp_body,
        out_shape=jax.ShapeDtypeStruct((B, 1), jnp.float32),
    )(vec, w1[:H], w1[H:], b1.reshape(1, MLP_DIM), w2,
      b2.reshape(1, MLP_DIM), w3.reshape(1, MLP_DIM), b3.reshape(1, 1))


def _prep_edges(ei_sol, ei_solv):
    def prep(row, fill):
        r = row.reshape(NS, E // NS)
        r = jnp.pad(r, ((0, 0), (0, EPT - E // NS)), constant_values=fill)
        return r.reshape(NS, NCHUNK, CHUNK)

    srci = jnp.stack([prep(ei_sol[0], 0), prep(ei_solv[0], 0)])
    srci2 = jnp.stack([2 * srci, 2 * srci + 1], axis=2)
    dsti = jnp.stack([prep(ei_sol[1], TRASH), prep(ei_solv[1], TRASH)])
    return srci2, dsti


def kernel(x_solute, x_solvent, edge_index_solute, edge_index_solvent,
           graph_ids, W_in_solute, W_msg_solute, W_in_solvent, W_msg_solvent,
           W1, b1, W2, b2, W3, b3):
    x_both = jnp.stack([x_solute, x_solvent])
    x_both = jnp.pad(x_both, ((0, 0), (0, NP - N), (0, 0)))
    wi = jnp.stack([W_in_solute, W_in_solvent])
    wm = jnp.stack([W_msg_solute, W_msg_solvent])
    srci2, dsti = _prep_edges(edge_index_solute, edge_index_solvent)
    zeros = jnp.zeros((NP, HH), jnp.float32)
    ids = jnp.pad(graph_ids, (0, NP - N), constant_values=B).reshape(NP, 1)

    h0 = _tc_in(x_both, wi)
    h = h0
    for _ in range(DEPTH):
        m = _segsum_both(h, srci2, dsti, zeros)
        h = _tc_update(m, h0, wm)
    vec = _tc_readout(h, ids)
    return _tc_mlp(vec, W1, b1, W2, b2, W3, b3)


# speedup vs baseline: 3.7961x; 1.1490x over previous
import functools

import jax
import jax.numpy as jnp
from jax import lax
from jax.experimental import pallas as pl
from jax.experimental.pallas import tpu as pltpu
from jax.experimental.pallas import tpu_sc as plsc

N = 10000
E = 320000
D = 128
H = 128
B = 128
MLP_DIM = 300
DEPTH = 3

NC = 2
NS = 16
CHUNK = 128
NCHUNK = 158
EPT = NCHUNK * CHUNK
TRASH = N
NP = 10240
BLK = 1024


HH = H // 2


@functools.cache
def _sc_seg_call():
    mesh = plsc.VectorSubcoreMesh(core_axis_name="c", subcore_axis_name="s")

    @functools.partial(
        pl.kernel,
        out_type=jax.ShapeDtypeStruct((NC, NP, H), jnp.float32),
        mesh=mesh,
        compiler_params=pltpu.CompilerParams(use_tc_tiling_on_sc=False),
        scratch_types=[
            pltpu.VMEM((2, NCHUNK, CHUNK), jnp.int32),
            pltpu.VMEM((NCHUNK, CHUNK), jnp.int32),
            pltpu.VMEM((2, CHUNK, HH), jnp.float32),
            pltpu.VMEM_SHARED((NP, HH), jnp.float32),
            pltpu.SemaphoreType.DMA,
            pltpu.SemaphoreType.DMA,
        ],
    )
    def sc_seg(h_hbm, srci_hbm, dsti_hbm, zeros_hbm, out_hbm,
               srcv, dstv, rows, acc, sem0, sem1):
        cid = lax.axis_index("c")
        sid = lax.axis_index("s")
        h_v = h_hbm.at[cid]
        out_v = out_hbm.at[cid]
        pltpu.sync_copy(srci_hbm.at[cid, sid], srcv)
        pltpu.sync_copy(dsti_hbm.at[cid, sid], dstv)
        rpt = NP // NS
        sems = (sem0, sem1)

        for half in range(2):
            col = pl.ds(half * HH, HH)
            pltpu.sync_copy(zeros_hbm.at[pl.ds(sid * rpt, rpt)],
                            acc.at[pl.ds(sid * rpt, rpt)])
            plsc.subcore_barrier()

            def gather(j, buf):
                return pltpu.make_async_copy(h_v.at[srcv.at[half, j]],
                                             rows.at[buf], sems[buf])

            gather(0, 0).start()

            @pl.loop(0, NCHUNK, step=2)
            def _(j):
                gather(j, 0).wait()
                gather(j + 1, 1).start()
                pltpu.sync_copy(rows.at[0], acc.at[dstv.at[j]], add=True)
                gather(j + 1, 1).wait()

                @pl.when(j + 2 < NCHUNK)
                def _():
                    gather(j + 2, 0).start()

                pltpu.sync_copy(rows.at[1], acc.at[dstv.at[j + 1]], add=True)

            plsc.subcore_barrier()
            pltpu.sync_copy(acc.at[pl.ds(sid * rpt, rpt)],
                            out_v.at[pl.ds(sid * rpt, rpt), col])

    return sc_seg


def _segsum_both(h, srci2, dsti, zeros):
    h2 = h.reshape(NC, 2 * NP, HH)
    return _sc_seg_call()(h2, srci2, dsti, zeros)


def _dot(a, b):
    return lax.dot_general(a, b, (((1,), (0,)), ((), ())),
                           precision=lax.Precision.HIGHEST,
                           preferred_element_type=jnp.float32)


def _tc_in_body(x_ref, w_ref, o_ref):
    o_ref[0] = jnp.maximum(_dot(x_ref[0], w_ref[0]), 0.0)


def _tc_in(x, wi):
    return pl.pallas_call(
        _tc_in_body,
        grid=(NC, NP // BLK),
        in_specs=[pl.BlockSpec((1, BLK, D), lambda m, i: (m, i, 0)),
                  pl.BlockSpec((1, D, H), lambda m, i: (m, 0, 0))],
        out_specs=pl.BlockSpec((1, BLK, H), lambda m, i: (m, i, 0)),
        out_shape=jax.ShapeDtypeStruct((NC, NP, H), jnp.float32),
    )(x, wi)


def _tc_update_body(m_ref, h0_ref, w_ref, o_ref):
    o_ref[0] = jnp.maximum(h0_ref[0] + _dot(m_ref[0], w_ref[0]), 0.0)


def _tc_update(m, h0, wm):
    return pl.pallas_call(
        _tc_update_body,
        grid=(NC, NP // BLK),
        in_specs=[pl.BlockSpec((1, BLK, H), lambda m, i: (m, i, 0)),
                  pl.BlockSpec((1, BLK, H), lambda m, i: (m, i, 0)),
                  pl.BlockSpec((1, H, H), lambda m, i: (m, 0, 0))],
        out_specs=pl.BlockSpec((1, BLK, H), lambda m, i: (m, i, 0)),
        out_shape=jax.ShapeDtypeStruct((NC, NP, H), jnp.float32),
    )(m, h0, wm)


def _tc_readout_body(h_ref, ids_ref, o_ref, sacc, cacc):
    i = pl.program_id(1)

    @pl.when(i == 0)
    def _():
        sacc[...] = jnp.zeros((B, H), jnp.float32)
        cacc[...] = jnp.zeros((B, H), jnp.float32)

    oh = (ids_ref[...] == lax.broadcasted_iota(jnp.int32, (BLK, B), 1))
    oh = oh.astype(jnp.float32)
    contract = (((0,), (0,)), ((), ()))
    sacc[...] += lax.dot_general(oh, h_ref[0], contract,
                                 precision=lax.Precision.HIGHEST,
                                 preferred_element_type=jnp.float32)
    cacc[...] += lax.dot_general(oh, jnp.ones((BLK, H), jnp.float32), contract,
                                 precision=lax.Precision.HIGHEST,
                                 preferred_element_type=jnp.float32)

    @pl.when(i == NP // BLK - 1)
    def _():
        o_ref[0] = sacc[...] / jnp.maximum(cacc[...], 1.0)


def _tc_readout(h, ids):
    return pl.pallas_call(
        _tc_readout_body,
        grid=(NC, NP // BLK),
        in_specs=[pl.BlockSpec((1, BLK, H), lambda m, i: (m, i, 0)),
                  pl.BlockSpec((BLK, 1), lambda m, i: (i, 0))],
        out_specs=pl.BlockSpec((1, B, H), lambda m, i: (m, 0, 0)),
        out_shape=jax.ShapeDtypeStruct((NC, B, H), jnp.float32),
        scratch_shapes=[pltpu.VMEM((B, H), jnp.float32),
                        pltpu.VMEM((B, H), jnp.float32)],
    )(h, ids)


def _tc_mlp_body(v_ref, w1a_ref, w1b_ref, b1_ref, w2_ref, b2_ref, w3_ref,
                 b3_ref, o_ref):
    h1 = jnp.maximum(_dot(v_ref[0], w1a_ref[...]) +
                     _dot(v_ref[1], w1b_ref[...]) + b1_ref[...], 0.0)
    h2 = jnp.maximum(_dot(h1, w2_ref[...]) + b2_ref[...], 0.0)
    o_ref[...] = jnp.sum(h2 * w3_ref[...], axis=1, keepdims=True) + b3_ref[...]


def _tc_mlp(vec, w1, b1, w2, b2, w3, b3):
    return pl.pallas_call(
        _tc_mlp_body,
        out_shape=jax.ShapeDtypeStruct((B, 1), jnp.float32),
    )(vec, w1[:H], w1[H:], b1.reshape(1, MLP_DIM), w2,
      b2.reshape(1, MLP_DIM), w3.reshape(1, MLP_DIM), b3.reshape(1, 1))


def _prep_edges(ei_sol, ei_solv):
    def prep(row, fill):
        r = row.reshape(NS, E // NS)
        r = jnp.pad(r, ((0, 0), (0, EPT - E // NS)), constant_values=fill)
        return r.reshape(NS, NCHUNK, CHUNK)

    srci = jnp.stack([prep(ei_sol[0], 0), prep(ei_solv[0], 0)])
    srci2 = jnp.stack([2 * srci, 2 * srci + 1], axis=2)
    dsti = jnp.stack([prep(ei_sol[1], TRASH), prep(ei_solv[1], TRASH)])
    return srci2, dsti


def kernel(x_solute, x_solvent, edge_index_solute, edge_index_solvent,
           graph_ids, W_in_solute, W_msg_solute, W_in_solvent, W_msg_solvent,
           W1, b1, W2, b2, W3, b3):
    x_both = jnp.stack([x_solute, x_solvent])
    x_both = jnp.pad(x_both, ((0, 0), (0, NP - N), (0, 0)))
    wi = jnp.stack([W_in_solute, W_in_solvent])
    wm = jnp.stack([W_msg_solute, W_msg_solvent])
    srci, dsti = _prep_edges(edge_index_solute, edge_index_solvent)
    zeros = jnp.zeros((NP, HH), jnp.float32)
    ids = jnp.pad(graph_ids, (0, NP - N), constant_values=B).reshape(NP, 1)

    h0 = _tc_in(x_both, wi)
    h = h0
    for _ in range(DEPTH):
        m = _segsum_both(h, srci, dsti, zeros)
        h = _tc_update(m, h0, wm)
    vec = _tc_readout(h, ids)
    return _tc_mlp(vec, W1, b1, W2, b2, W3, b3)
